# baseline (speedup 1.0000x reference)
import functools

import jax
import jax.numpy as jnp
from jax.experimental import pallas as pl
from jax.experimental.pallas import tpu as pltpu

_LO = 64


def _node_kernel(at0_ref, at1_ref, xt_ref, w1t_ref, b1_ref, w2t_ref, b2_ref,
                 wheadt_ref, y_ref):
    f32 = jnp.float32
    n = at0_ref.shape[0]
    a = at0_ref[...] + at1_ref[0:n, :]
    row = jax.lax.broadcasted_iota(jnp.int32, (n, n), 0)
    col = jax.lax.broadcasted_iota(jnp.int32, (n, n), 1)
    a = jnp.where(row == col, a + 1.0, a)
    dinv = jax.lax.rsqrt(jnp.sum(a, axis=0, keepdims=True))
    xa = jnp.dot(xt_ref[...] * dinv, a, preferred_element_type=f32) * dinv
    h1 = jnp.maximum(
        jnp.dot(w1t_ref[...], xa, preferred_element_type=f32) + b1_ref[...], 0.0)
    h1w = jnp.dot(w2t_ref[...], h1, preferred_element_type=f32) * dinv
    h2 = jnp.maximum(
        jnp.dot(h1w, a, preferred_element_type=f32) * dinv + b2_ref[...], 0.0)
    y_ref[...] = jnp.dot(wheadt_ref[...], h2, preferred_element_type=f32)


def _edge_kernel(wt_ref, src_ref, dst_ref, attr_ref, wattr_ref, blin_ref,
                 out_ref):
    f32 = jnp.float32
    te = src_ref.shape[1]
    hi = wt_ref.shape[0] // 4

    s = src_ref[0]
    d = dst_ref[0]
    sub_lo = jax.lax.broadcasted_iota(jnp.int32, (_LO, te), 0)
    oh_s = (sub_lo == (s & (_LO - 1))).astype(f32)
    oh_d = (sub_lo == (d & (_LO - 1))).astype(f32)
    g = jnp.concatenate([oh_s, oh_d], axis=0)
    t = jnp.dot(wt_ref[...], g, preferred_element_type=f32)

    sub_hi = jax.lax.broadcasted_iota(jnp.int32, (hi, te), 0)
    hs = sub_hi == (s >> 6)
    hd = sub_hi == (d >> 6)
    zero = jnp.zeros((), f32)
    a = attr_ref[...]
    o0 = (jnp.sum(jnp.where(hs, t[0:hi], zero), axis=0, keepdims=True)
          + jnp.sum(jnp.where(hd, t[2 * hi:3 * hi], zero), axis=0, keepdims=True)
          + jnp.sum(a * wattr_ref[:, 0:1], axis=0, keepdims=True))
    o1 = (jnp.sum(jnp.where(hs, t[hi:2 * hi], zero), axis=0, keepdims=True)
          + jnp.sum(jnp.where(hd, t[3 * hi:4 * hi], zero), axis=0, keepdims=True)
          + jnp.sum(a * wattr_ref[:, 1:2], axis=0, keepdims=True))
    out_ref[...] = jnp.concatenate([o0, o1], axis=0) + blin_ref[...]


@functools.partial(jax.jit, static_argnames=("edge_tile",))
def _forward(x, edge_index, edge_attr, W1, b1, W2, b2, Wlin, blin,
             edge_tile=4096):
    f32 = jnp.float32
    N, F_IN = x.shape
    E = edge_index.shape[1]
    H1 = W1.shape[1]
    H2 = W2.shape[1]
    A_DIM = edge_attr.shape[1]
    HI = -(-N // _LO)

    src, dst = edge_index[0].astype(jnp.int32), edge_index[1].astype(jnp.int32)
    half = E // 2
    at0 = jnp.zeros((N, N), f32).at[src[:half], dst[:half]].add(1.0)
    at1 = jnp.zeros((N + 8, N), f32).at[src[half:], dst[half:]].add(1.0)

    w_src = Wlin[:H2]
    w_attr = Wlin[H2:H2 + A_DIM]
    w_dst = Wlin[H2 + A_DIM:]
    w_head_t = jnp.concatenate([w_src, w_dst], axis=1).T.astype(f32)

    y = pl.pallas_call(
        _node_kernel,
        out_shape=jax.ShapeDtypeStruct((4, N), f32),
        grid=(1,),
        in_specs=[
            pl.BlockSpec((N, N), lambda i: (0, 0)),
            pl.BlockSpec((N + 8, N), lambda i: (0, 0)),
            pl.BlockSpec((F_IN, N), lambda i: (0, 0)),
            pl.BlockSpec((H1, F_IN), lambda i: (0, 0)),
            pl.BlockSpec((H1, 1), lambda i: (0, 0)),
            pl.BlockSpec((H2, H1), lambda i: (0, 0)),
            pl.BlockSpec((H2, 1), lambda i: (0, 0)),
            pl.BlockSpec((4, H2), lambda i: (0, 0)),
        ],
        out_specs=pl.BlockSpec((4, N), lambda i: (0, 0)),
        compiler_params=pltpu.CompilerParams(dimension_semantics=("arbitrary",)),
    )(at0, at1, x.T.astype(f32), W1.T.astype(f32), b1.reshape(-1, 1).astype(f32),
      W2.T.astype(f32), b2.reshape(-1, 1).astype(f32), w_head_t)

    y_pad = jnp.zeros((4, HI * _LO), f32).at[:, :N].set(y) if HI * _LO != N else y
    wt = (jnp.zeros((4 * HI, 2 * _LO), f32)
          .at[:2 * HI, :_LO].set(y_pad[0:2].reshape(2 * HI, _LO))
          .at[2 * HI:, _LO:].set(y_pad[2:4].reshape(2 * HI, _LO)))

    e_pad = -(-E // (2 * edge_tile)) * (2 * edge_tile)
    if e_pad != E:
        ei3 = (jnp.zeros((2, 1, e_pad), jnp.int32)
               .at[0, 0, :E].set(src).at[1, 0, :E].set(dst))
        attr_t = jnp.zeros((A_DIM, e_pad), f32).at[:, :E].set(edge_attr.T.astype(f32))
    else:
        ei3 = edge_index.astype(jnp.int32).reshape(2, 1, E)
        attr_t = edge_attr.T.astype(f32)

    nt = e_pad // (2 * edge_tile)
    out_t = pl.pallas_call(
        _edge_kernel,
        out_shape=jax.ShapeDtypeStruct((2, e_pad), f32),
        grid=(2, nt),
        in_specs=[
            pl.BlockSpec((4 * HI, 2 * _LO), lambda c, t: (0, 0)),
            pl.BlockSpec((1, 1, edge_tile), lambda c, t: (0, 0, c * nt + t)),
            pl.BlockSpec((1, 1, edge_tile), lambda c, t: (1, 0, c * nt + t)),
            pl.BlockSpec((A_DIM, edge_tile), lambda c, t: (0, c * nt + t)),
            pl.BlockSpec((A_DIM, 2), lambda c, t: (0, 0)),
            pl.BlockSpec((2, 1), lambda c, t: (0, 0)),
        ],
        out_specs=pl.BlockSpec((2, edge_tile), lambda c, t: (0, c * nt + t)),
        compiler_params=pltpu.CompilerParams(
            dimension_semantics=("parallel", "arbitrary")),
    )(wt, ei3, ei3, attr_t, w_attr.astype(f32), blin.reshape(2, 1).astype(f32))

    out_f = out_t if e_pad == E else out_t[:, :E]
    return out_f.T


def kernel(x, edge_index, edge_attr, W1, b1, W2, b2, Wlin, blin):
    return _forward(x, edge_index, edge_attr, W1, b1, W2, b2, Wlin, blin,
                    edge_tile=16384)


# speedup vs baseline: 1.0006x; 1.0006x over previous
import functools

import jax
import jax.numpy as jnp
from jax.experimental import pallas as pl
from jax.experimental.pallas import tpu as pltpu

_LO = 64


def _node_kernel(at0_ref, at1_ref, xt_ref, w1t_ref, b1_ref, w2t_ref, b2_ref,
                 wheadt_ref, y_ref):
    f32 = jnp.float32
    n = at0_ref.shape[0]
    a = at0_ref[...] + at1_ref[0:n, :]
    row = jax.lax.broadcasted_iota(jnp.int32, (n, n), 0)
    col = jax.lax.broadcasted_iota(jnp.int32, (n, n), 1)
    a = jnp.where(row == col, a + 1.0, a)
    dinv = jax.lax.rsqrt(jnp.sum(a, axis=0, keepdims=True))
    xa = jnp.dot(xt_ref[...] * dinv, a, preferred_element_type=f32) * dinv
    h1 = jnp.maximum(
        jnp.dot(w1t_ref[...], xa, preferred_element_type=f32) + b1_ref[...], 0.0)
    h1w = jnp.dot(w2t_ref[...], h1, preferred_element_type=f32) * dinv
    h2 = jnp.maximum(
        jnp.dot(h1w, a, preferred_element_type=f32) * dinv + b2_ref[...], 0.0)
    y_ref[...] = jnp.dot(wheadt_ref[...], h2, preferred_element_type=f32)


def _edge_kernel(wt_ref, src_ref, dst_ref, attr_ref, wattr_ref, blin_ref,
                 out_ref):
    f32 = jnp.float32
    te = src_ref.shape[1]
    hi = wt_ref.shape[0] // 4

    s = src_ref[0]
    d = dst_ref[0]
    sub_lo = jax.lax.broadcasted_iota(jnp.int32, (_LO, te), 0)
    oh_s = (sub_lo == (s & (_LO - 1))).astype(f32)
    oh_d = (sub_lo == (d & (_LO - 1))).astype(f32)
    g = jnp.concatenate([oh_s, oh_d], axis=0)
    t = jnp.dot(wt_ref[...], g, preferred_element_type=f32)

    sub_hi = jax.lax.broadcasted_iota(jnp.int32, (hi, te), 0)
    hs = sub_hi == (s >> 6)
    hd = sub_hi == (d >> 6)
    zero = jnp.zeros((), f32)
    a = attr_ref[...]
    o0 = (jnp.sum(jnp.where(hs, t[0:hi], zero), axis=0, keepdims=True)
          + jnp.sum(jnp.where(hd, t[2 * hi:3 * hi], zero), axis=0, keepdims=True)
          + jnp.sum(a * wattr_ref[:, 0:1], axis=0, keepdims=True))
    o1 = (jnp.sum(jnp.where(hs, t[hi:2 * hi], zero), axis=0, keepdims=True)
          + jnp.sum(jnp.where(hd, t[3 * hi:4 * hi], zero), axis=0, keepdims=True)
          + jnp.sum(a * wattr_ref[:, 1:2], axis=0, keepdims=True))
    out_ref[...] = jnp.concatenate([o0, o1], axis=0) + blin_ref[...]


@functools.partial(jax.jit, static_argnames=("edge_tile",))
def _forward(x, edge_index, edge_attr, W1, b1, W2, b2, Wlin, blin,
             edge_tile=4096):
    f32 = jnp.float32
    N, F_IN = x.shape
    E = edge_index.shape[1]
    H1 = W1.shape[1]
    H2 = W2.shape[1]
    A_DIM = edge_attr.shape[1]
    HI = -(-N // _LO)

    src, dst = edge_index[0].astype(jnp.int32), edge_index[1].astype(jnp.int32)
    half = E // 2
    at0 = jnp.zeros((N, N), f32).at[src[:half], dst[:half]].add(1.0)
    at1 = jnp.zeros((N + 8, N), f32).at[src[half:], dst[half:]].add(1.0)

    w_src = Wlin[:H2]
    w_attr = Wlin[H2:H2 + A_DIM]
    w_dst = Wlin[H2 + A_DIM:]
    w_head_t = jnp.concatenate([w_src, w_dst], axis=1).T.astype(f32)

    y = pl.pallas_call(
        _node_kernel,
        out_shape=jax.ShapeDtypeStruct((4, N), f32),
        grid=(1,),
        in_specs=[
            pl.BlockSpec((N, N), lambda i: (0, 0)),
            pl.BlockSpec((N + 8, N), lambda i: (0, 0)),
            pl.BlockSpec((F_IN, N), lambda i: (0, 0)),
            pl.BlockSpec((H1, F_IN), lambda i: (0, 0)),
            pl.BlockSpec((H1, 1), lambda i: (0, 0)),
            pl.BlockSpec((H2, H1), lambda i: (0, 0)),
            pl.BlockSpec((H2, 1), lambda i: (0, 0)),
            pl.BlockSpec((4, H2), lambda i: (0, 0)),
        ],
        out_specs=pl.BlockSpec((4, N), lambda i: (0, 0)),
        compiler_params=pltpu.CompilerParams(dimension_semantics=("arbitrary",)),
    )(at0, at1, x.T.astype(f32), W1.T.astype(f32), b1.reshape(-1, 1).astype(f32),
      W2.T.astype(f32), b2.reshape(-1, 1).astype(f32), w_head_t)

    y_pad = jnp.zeros((4, HI * _LO), f32).at[:, :N].set(y) if HI * _LO != N else y
    wt = (jnp.zeros((4 * HI, 2 * _LO), f32)
          .at[:2 * HI, :_LO].set(y_pad[0:2].reshape(2 * HI, _LO))
          .at[2 * HI:, _LO:].set(y_pad[2:4].reshape(2 * HI, _LO)))

    e_pad = -(-E // (2 * edge_tile)) * (2 * edge_tile)
    if e_pad != E:
        ei3 = (jnp.zeros((2, 1, e_pad), jnp.int32)
               .at[0, 0, :E].set(src).at[1, 0, :E].set(dst))
        attr_t = jnp.zeros((A_DIM, e_pad), f32).at[:, :E].set(edge_attr.T.astype(f32))
    else:
        ei3 = edge_index.astype(jnp.int32).reshape(2, 1, E)
        attr_t = edge_attr.T.astype(f32)

    nt = e_pad // (2 * edge_tile)
    out_t = pl.pallas_call(
        _edge_kernel,
        out_shape=jax.ShapeDtypeStruct((2, e_pad), f32),
        grid=(2, nt),
        in_specs=[
            pl.BlockSpec((4 * HI, 2 * _LO), lambda c, t: (0, 0)),
            pl.BlockSpec((1, 1, edge_tile), lambda c, t: (0, 0, c * nt + t)),
            pl.BlockSpec((1, 1, edge_tile), lambda c, t: (1, 0, c * nt + t)),
            pl.BlockSpec((A_DIM, edge_tile), lambda c, t: (0, c * nt + t)),
            pl.BlockSpec((A_DIM, 2), lambda c, t: (0, 0)),
            pl.BlockSpec((2, 1), lambda c, t: (0, 0)),
        ],
        out_specs=pl.BlockSpec((2, edge_tile), lambda c, t: (0, c * nt + t)),
        compiler_params=pltpu.CompilerParams(
            dimension_semantics=("parallel", "arbitrary")),
    )(wt, ei3, ei3, attr_t, w_attr.astype(f32), blin.reshape(2, 1).astype(f32))

    out_f = out_t if e_pad == E else out_t[:, :E]
    return out_f.T


def kernel(x, edge_index, edge_attr, W1, b1, W2, b2, Wlin, blin):
    return _forward(x, edge_index, edge_attr, W1, b1, W2, b2, Wlin, blin,
                    edge_tile=8192)
